# baseline (speedup 1.0000x reference)
import functools

import jax
import jax.numpy as jnp
from jax import lax
from jax.experimental import pallas as pl
from jax.experimental.pallas import tpu as pltpu
from jax.experimental.pallas import tpu_sc as plsc

_N = 10000
_E = 320000
_D = 128
_NSUB = 16
_NPAD = 10240
_EMAIN = 19968
_ETAIL = _E - _NSUB * _EMAIN
_EC = _EMAIN + _ETAIL
_NC = _NPAD // _NSUB
_L = 16
_DUMMY = _NPAD - 1
_UNROLL = 8


def _rsqrt16(x):
    i = lax.bitcast_convert_type(x, jnp.int32)
    i = jnp.int32(0x5F3759DF) - (i >> 1)
    y = lax.bitcast_convert_type(i, jnp.float32)
    for _ in range(3):
        y = y * (jnp.float32(1.5) - jnp.float32(0.5) * x * y * y)
    return y


def _sc_node_weights(ei_hbm, w_hbm,
                     ei_v, deg_l, s_l, dis_l, red_v, node_w, slot_sp, dis_sp):
    wid = lax.axis_index("s")
    ebase = wid * _EMAIN
    nbase = wid * _NC
    ones = jnp.full((_L,), 1.0, jnp.float32)

    def _fill_dummy(i, c):
        sl = pl.ds(_EMAIN + i * _L, _L)
        ei_v[0, sl] = jnp.full((_L,), _DUMMY, jnp.int32)
        ei_v[1, sl] = jnp.full((_L,), _DUMMY, jnp.int32)
        return c
    lax.fori_loop(0, _ETAIL // _L, _fill_dummy, 0)

    pltpu.sync_copy(ei_hbm.at[:, pl.ds(ebase, _EMAIN)],
                    ei_v.at[:, pl.ds(0, _EMAIN)])

    @pl.when(wid == _NSUB - 1)
    def _tail():
        pltpu.sync_copy(ei_hbm.at[:, pl.ds(_NSUB * _EMAIN, _ETAIL)],
                        ei_v.at[:, pl.ds(_EMAIN, _ETAIL)])

    def _zero(i, c):
        sl = pl.ds(i * _L, _L)
        deg_l[sl] = jnp.zeros((_L,), jnp.float32)
        s_l[sl] = jnp.zeros((_L,), jnp.float32)
        return c
    lax.fori_loop(0, _NPAD // _L, _zero, 0)

    def _hist(i, c):
        for u in range(_UNROLL):
            k = (i * _UNROLL + u) * _L
            idx = ei_v[1, pl.ds(k, _L)]
            plsc.addupdate_scatter(deg_l, [idx], ones)
        return c
    lax.fori_loop(0, _EC // (_L * _UNROLL), _hist, 0)

    pltpu.sync_copy(deg_l, slot_sp.at[wid])
    plsc.subcore_barrier()
    pltpu.sync_copy(slot_sp.at[:, pl.ds(nbase, _NC)], red_v)

    def _mkdis(c, carry):
        sl = pl.ds(c * _L, _L)
        acc = jnp.full((_L,), 1.0, jnp.float32)
        for j in range(_NSUB):
            acc = acc + red_v[j, sl]
        node_w[sl] = _rsqrt16(acc)
        return carry
    lax.fori_loop(0, _NC // _L, _mkdis, 0)
    pltpu.sync_copy(node_w, dis_sp.at[pl.ds(nbase, _NC)])
    plsc.subcore_barrier()

    pltpu.sync_copy(dis_sp, dis_l)

    def _seg(i, c):
        for u in range(_UNROLL):
            k = (i * _UNROLL + u) * _L
            d_idx = ei_v[1, pl.ds(k, _L)]
            s_idx = ei_v[0, pl.ds(k, _L)]
            g = plsc.load_gather(dis_l, [d_idx])
            plsc.addupdate_scatter(s_l, [s_idx], g)
        return c
    lax.fori_loop(0, _EC // (_L * _UNROLL), _seg, 0)

    pltpu.sync_copy(s_l, slot_sp.at[wid])
    plsc.subcore_barrier()
    pltpu.sync_copy(slot_sp.at[:, pl.ds(nbase, _NC)], red_v)

    def _wfin(c, carry):
        sl = pl.ds(c * _L, _L)
        acc = red_v[0, sl]
        for j in range(1, _NSUB):
            acc = acc + red_v[j, sl]
        d = node_w[sl]
        node_w[sl] = d * (acc + d)
        return carry
    lax.fori_loop(0, _NC // _L, _wfin, 0)
    pltpu.sync_copy(node_w, w_hbm.at[0, pl.ds(nbase, _NC)])


_sc_kernel = functools.partial(
    pl.kernel,
    out_type=jax.ShapeDtypeStruct((1, _NPAD), jnp.float32),
    mesh=plsc.VectorSubcoreMesh(core_axis_name="c", subcore_axis_name="s",
                                num_cores=1),
    compiler_params=pltpu.CompilerParams(needs_layout_passes=False),
    scratch_types=[
        pltpu.VMEM((2, _EC), jnp.int32),
        pltpu.VMEM((_NPAD,), jnp.float32),
        pltpu.VMEM((_NPAD,), jnp.float32),
        pltpu.VMEM((_NPAD,), jnp.float32),
        pltpu.VMEM((_NSUB, _NC), jnp.float32),
        pltpu.VMEM((_NC,), jnp.float32),
        pltpu.VMEM_SHARED((_NSUB, _NPAD), jnp.float32),
        pltpu.VMEM_SHARED((_NPAD,), jnp.float32),
    ],
)(_sc_node_weights)


def _tc_body(x_ref, wa_ref, aa_ref, w_ref, wg_ref, bg_ref, wo_ref, bo_ref,
             o_ref):
    x = x_ref[...]
    wh = jnp.dot(x, wa_ref[...], preferred_element_type=jnp.float32)
    e = jnp.dot(wh, aa_ref[...], preferred_element_type=jnp.float32)
    e = jnp.where(e > 0, e, jnp.float32(0.2) * e)
    p = jnp.exp(e - jnp.max(e))
    attn = p * (jnp.float32(1.0) / jnp.sum(p))
    z = attn * wh
    h1 = jnp.where(z > 0, z, jnp.exp(z) - jnp.float32(1.0))
    v = jnp.dot(w_ref[...][:, :_N], h1,
                preferred_element_type=jnp.float32)
    g = (jnp.dot(v, wg_ref[...], preferred_element_type=jnp.float32)
         * jnp.float32(1.0 / _N) + bg_ref[...])
    r = (jnp.dot(g, wo_ref[...], preferred_element_type=jnp.float32)
         + bo_ref[...])
    r = r - jnp.max(r)
    pr = jnp.exp(r)
    o_ref[...] = pr * (jnp.float32(1.0) / jnp.sum(pr))


def kernel(features1, edge_index1, edgesAttr1, adjacency1, node2node_features1,
           W_att, a_att, W_gcn, b_gcn, W_out, b_out):
    w_row = _sc_kernel(edge_index1)
    out = pl.pallas_call(
        _tc_body,
        out_shape=jax.ShapeDtypeStruct((1, 2), jnp.float32),
    )(features1, W_att, a_att, w_row, W_gcn, b_gcn.reshape(1, _D), W_out,
      b_out.reshape(1, 2))
    return out


# speedup vs baseline: 1.4048x; 1.4048x over previous
import functools

import jax
import jax.numpy as jnp
from jax import lax
from jax.experimental import pallas as pl
from jax.experimental.pallas import tpu as pltpu
from jax.experimental.pallas import tpu_sc as plsc

_N = 10000
_E = 320000
_D = 128
_NSUB = 16
_NPAD = 10240
_EMAIN = 19968
_ETAIL = _E - _NSUB * _EMAIN
_EC = _EMAIN + _ETAIL
_NC = _NPAD // _NSUB
_L = 16
_DUMMY = _NPAD - 1
_UNROLL = 8


def _rsqrt16(x):
    i = lax.bitcast_convert_type(x, jnp.int32)
    i = jnp.int32(0x5F3759DF) - (i >> 1)
    y = lax.bitcast_convert_type(i, jnp.float32)
    for _ in range(3):
        y = y * (jnp.float32(1.5) - jnp.float32(0.5) * x * y * y)
    return y


def _sc_node_weights(ei_hbm, w_hbm,
                     ei_v, deg_l, s_l, dis_l, red_v, node_w, slot_sp, dis_sp):
    wid = lax.axis_index("s")
    ebase = wid * _EMAIN
    nbase = wid * _NC
    ones = jnp.full((_L,), 1.0, jnp.float32)

    @plsc.parallel_loop(0, _ETAIL // _L, unroll=8)
    def _fill_dummy(i):
        sl = pl.ds(_EMAIN + i * _L, _L)
        ei_v[0, sl] = jnp.full((_L,), _DUMMY, jnp.int32)
        ei_v[1, sl] = jnp.full((_L,), _DUMMY, jnp.int32)

    pltpu.sync_copy(ei_hbm.at[:, pl.ds(ebase, _EMAIN)],
                    ei_v.at[:, pl.ds(0, _EMAIN)])

    @pl.when(wid == _NSUB - 1)
    def _tail():
        pltpu.sync_copy(ei_hbm.at[:, pl.ds(_NSUB * _EMAIN, _ETAIL)],
                        ei_v.at[:, pl.ds(_EMAIN, _ETAIL)])

    @plsc.parallel_loop(0, _NPAD // _L, unroll=8)
    def _zero(i):
        sl = pl.ds(i * _L, _L)
        deg_l[sl] = jnp.zeros((_L,), jnp.float32)
        s_l[sl] = jnp.zeros((_L,), jnp.float32)

    @plsc.parallel_loop(0, _EC // _L, unroll=_UNROLL)
    def _hist(i):
        idx = ei_v[1, pl.ds(i * _L, _L)]
        plsc.addupdate_scatter(deg_l, [idx], ones)

    pltpu.sync_copy(deg_l, slot_sp.at[wid])
    plsc.subcore_barrier()
    pltpu.sync_copy(slot_sp.at[:, pl.ds(nbase, _NC)], red_v)

    @plsc.parallel_loop(0, _NC // _L, unroll=4)
    def _mkdis(c):
        sl = pl.ds(c * _L, _L)
        acc = jnp.full((_L,), 1.0, jnp.float32)
        for j in range(_NSUB):
            acc = acc + red_v[j, sl]
        node_w[sl] = _rsqrt16(acc)
    pltpu.sync_copy(node_w, dis_sp.at[pl.ds(nbase, _NC)])
    plsc.subcore_barrier()

    pltpu.sync_copy(dis_sp, dis_l)

    @plsc.parallel_loop(0, _EC // _L, unroll=_UNROLL)
    def _seg(i):
        d_idx = ei_v[1, pl.ds(i * _L, _L)]
        s_idx = ei_v[0, pl.ds(i * _L, _L)]
        g = plsc.load_gather(dis_l, [d_idx])
        plsc.addupdate_scatter(s_l, [s_idx], g)

    pltpu.sync_copy(s_l, slot_sp.at[wid])
    plsc.subcore_barrier()
    pltpu.sync_copy(slot_sp.at[:, pl.ds(nbase, _NC)], red_v)

    @plsc.parallel_loop(0, _NC // _L, unroll=4)
    def _wfin(c):
        sl = pl.ds(c * _L, _L)
        acc = red_v[0, sl]
        for j in range(1, _NSUB):
            acc = acc + red_v[j, sl]
        d = node_w[sl]
        node_w[sl] = d * (acc + d)
    pltpu.sync_copy(node_w, w_hbm.at[0, pl.ds(nbase, _NC)])


_sc_kernel = functools.partial(
    pl.kernel,
    out_type=jax.ShapeDtypeStruct((1, _NPAD), jnp.float32),
    mesh=plsc.VectorSubcoreMesh(core_axis_name="c", subcore_axis_name="s",
                                num_cores=1),
    compiler_params=pltpu.CompilerParams(needs_layout_passes=False),
    scratch_types=[
        pltpu.VMEM((2, _EC), jnp.int32),
        pltpu.VMEM((_NPAD,), jnp.float32),
        pltpu.VMEM((_NPAD,), jnp.float32),
        pltpu.VMEM((_NPAD,), jnp.float32),
        pltpu.VMEM((_NSUB, _NC), jnp.float32),
        pltpu.VMEM((_NC,), jnp.float32),
        pltpu.VMEM_SHARED((_NSUB, _NPAD), jnp.float32),
        pltpu.VMEM_SHARED((_NPAD,), jnp.float32),
    ],
)(_sc_node_weights)


def _tc_body(x_ref, wa_ref, aa_ref, w_ref, wg_ref, bg_ref, wo_ref, bo_ref,
             o_ref):
    x = x_ref[...]
    wh = jnp.dot(x, wa_ref[...], preferred_element_type=jnp.float32)
    e = jnp.dot(wh, aa_ref[...], preferred_element_type=jnp.float32)
    e = jnp.where(e > 0, e, jnp.float32(0.2) * e)
    p = jnp.exp(e - jnp.max(e))
    attn = p * (jnp.float32(1.0) / jnp.sum(p))
    z = attn * wh
    h1 = jnp.where(z > 0, z, jnp.exp(z) - jnp.float32(1.0))
    v = jnp.dot(w_ref[...][:, :_N], h1,
                preferred_element_type=jnp.float32)
    g = (jnp.dot(v, wg_ref[...], preferred_element_type=jnp.float32)
         * jnp.float32(1.0 / _N) + bg_ref[...])
    r = (jnp.dot(g, wo_ref[...], preferred_element_type=jnp.float32)
         + bo_ref[...])
    r = r - jnp.max(r)
    pr = jnp.exp(r)
    o_ref[...] = pr * (jnp.float32(1.0) / jnp.sum(pr))


def kernel(features1, edge_index1, edgesAttr1, adjacency1, node2node_features1,
           W_att, a_att, W_gcn, b_gcn, W_out, b_out):
    w_row = _sc_kernel(edge_index1)
    out = pl.pallas_call(
        _tc_body,
        out_shape=jax.ShapeDtypeStruct((1, 2), jnp.float32),
    )(features1, W_att, a_att, w_row, W_gcn, b_gcn.reshape(1, _D), W_out,
      b_out.reshape(1, 2))
    return out


# speedup vs baseline: 1.5619x; 1.1118x over previous
import functools

import jax
import jax.numpy as jnp
from jax import lax
from jax.experimental import pallas as pl
from jax.experimental.pallas import tpu as pltpu
from jax.experimental.pallas import tpu_sc as plsc

_N = 10000
_E = 320000
_D = 128
_NSUB = 16
_NPAD = 10240
_EMAIN = 19968
_ETAIL = _E - _NSUB * _EMAIN
_EC = _EMAIN + _ETAIL
_NC = _NPAD // _NSUB
_L = 16
_DUMMY = _NPAD - 1
_UNROLL = 8


def _rsqrt16(x):
    i = lax.bitcast_convert_type(x, jnp.int32)
    i = jnp.int32(0x5F3759DF) - (i >> 1)
    y = lax.bitcast_convert_type(i, jnp.float32)
    for _ in range(3):
        y = y * (jnp.float32(1.5) - jnp.float32(0.5) * x * y * y)
    return y


def _sc_node_weights(ei_hbm, w_hbm,
                     ei_v, deg_l, s_l, dis_l, red_v, node_w, slot_sp, dis_sp):
    wid = lax.axis_index("s")
    ebase = wid * _EMAIN
    nbase = wid * _NC
    ones = jnp.full((_L,), 1.0, jnp.float32)

    @plsc.parallel_loop(0, _ETAIL // _L, unroll=8)
    def _fill_dummy(i):
        sl = pl.ds(_EMAIN + i * _L, _L)
        ei_v[0, sl] = jnp.full((_L,), _DUMMY, jnp.int32)
        ei_v[1, sl] = jnp.full((_L,), _DUMMY, jnp.int32)

    pltpu.sync_copy(ei_hbm.at[:, pl.ds(ebase, _EMAIN)],
                    ei_v.at[:, pl.ds(0, _EMAIN)])

    @pl.when(wid == _NSUB - 1)
    def _tail():
        pltpu.sync_copy(ei_hbm.at[:, pl.ds(_NSUB * _EMAIN, _ETAIL)],
                        ei_v.at[:, pl.ds(_EMAIN, _ETAIL)])

    @plsc.parallel_loop(0, _NPAD // _L, unroll=8)
    def _zero(i):
        sl = pl.ds(i * _L, _L)
        deg_l[sl] = jnp.zeros((_L,), jnp.float32)
        s_l[sl] = jnp.zeros((_L,), jnp.float32)

    @plsc.parallel_loop(0, _EC // _L, unroll=_UNROLL)
    def _hist(i):
        idx = ei_v[1, pl.ds(i * _L, _L)]
        plsc.addupdate_scatter(deg_l, [idx], ones)

    pltpu.sync_copy(deg_l, slot_sp.at[wid])
    plsc.subcore_barrier()
    pltpu.sync_copy(slot_sp.at[:, pl.ds(nbase, _NC)], red_v)

    @plsc.parallel_loop(0, _NC // _L, unroll=4)
    def _mkdis(c):
        sl = pl.ds(c * _L, _L)
        acc = jnp.full((_L,), 1.0, jnp.float32)
        for j in range(_NSUB):
            acc = acc + red_v[j, sl]
        node_w[sl] = _rsqrt16(acc)
    pltpu.sync_copy(node_w, dis_sp.at[pl.ds(nbase, _NC)])
    plsc.subcore_barrier()

    pltpu.sync_copy(dis_sp, dis_l)

    @plsc.parallel_loop(0, _EC // _L, unroll=_UNROLL)
    def _seg(i):
        d_idx = ei_v[1, pl.ds(i * _L, _L)]
        s_idx = ei_v[0, pl.ds(i * _L, _L)]
        g = plsc.load_gather(dis_l, [d_idx])
        plsc.addupdate_scatter(s_l, [s_idx], g)

    pltpu.sync_copy(s_l, slot_sp.at[wid])
    plsc.subcore_barrier()
    pltpu.sync_copy(slot_sp.at[:, pl.ds(nbase, _NC)], red_v)

    @plsc.parallel_loop(0, _NC // _L, unroll=4)
    def _wfin(c):
        sl = pl.ds(c * _L, _L)
        acc = red_v[0, sl]
        for j in range(1, _NSUB):
            acc = acc + red_v[j, sl]
        d = node_w[sl]
        node_w[sl] = d * (acc + d)
    pltpu.sync_copy(node_w, w_hbm.at[0, pl.ds(nbase, _NC)])


_sc_kernel = functools.partial(
    pl.kernel,
    out_type=jax.ShapeDtypeStruct((1, _NPAD), jnp.float32),
    mesh=plsc.VectorSubcoreMesh(core_axis_name="c", subcore_axis_name="s",
                                num_cores=1),
    compiler_params=pltpu.CompilerParams(needs_layout_passes=False),
    scratch_types=[
        pltpu.VMEM((2, _EC), jnp.int32),
        pltpu.VMEM((_NPAD,), jnp.float32),
        pltpu.VMEM((_NPAD,), jnp.float32),
        pltpu.VMEM((_NPAD,), jnp.float32),
        pltpu.VMEM((_NSUB, _NC), jnp.float32),
        pltpu.VMEM((_NC,), jnp.float32),
        pltpu.VMEM_SHARED((_NSUB, _NPAD), jnp.float32),
        pltpu.VMEM_SHARED((_NPAD,), jnp.float32),
    ],
)(_sc_node_weights)


def _tc_a_body(x_ref, wa_ref, aa_ref, h1_ref):
    x = x_ref[...]
    wh = jnp.dot(x, wa_ref[...], preferred_element_type=jnp.float32)
    e = jnp.dot(wh, aa_ref[...], preferred_element_type=jnp.float32)
    e = jnp.where(e > 0, e, jnp.float32(0.2) * e)
    p = jnp.exp(e - jnp.max(e))
    attn = p * (jnp.float32(1.0) / jnp.sum(p))
    z = attn * wh
    h1_ref[...] = jnp.where(z > 0, z, jnp.exp(z) - jnp.float32(1.0))


def _tc_b_body(h1_ref, w_ref, wg_ref, bg_ref, wo_ref, bo_ref, o_ref):
    v = jnp.dot(w_ref[...][:, :_N], h1_ref[...],
                preferred_element_type=jnp.float32)
    g = (jnp.dot(v, wg_ref[...], preferred_element_type=jnp.float32)
         * jnp.float32(1.0 / _N) + bg_ref[...])
    r = (jnp.dot(g, wo_ref[...], preferred_element_type=jnp.float32)
         + bo_ref[...])
    r = r - jnp.max(r)
    pr = jnp.exp(r)
    o_ref[...] = pr * (jnp.float32(1.0) / jnp.sum(pr))


def kernel(features1, edge_index1, edgesAttr1, adjacency1, node2node_features1,
           W_att, a_att, W_gcn, b_gcn, W_out, b_out):
    w_row = _sc_kernel(edge_index1)
    h1 = pl.pallas_call(
        _tc_a_body,
        out_shape=jax.ShapeDtypeStruct((_N, _D), jnp.float32),
    )(features1, W_att, a_att)
    out = pl.pallas_call(
        _tc_b_body,
        out_shape=jax.ShapeDtypeStruct((1, 2), jnp.float32),
    )(h1, w_row, W_gcn, b_gcn.reshape(1, _D), W_out, b_out.reshape(1, 2))
    return out


# speedup vs baseline: 1.6288x; 1.0429x over previous
import functools

import jax
import jax.numpy as jnp
from jax import lax
from jax.experimental import pallas as pl
from jax.experimental.pallas import tpu as pltpu
from jax.experimental.pallas import tpu_sc as plsc

_N = 10000
_E = 320000
_D = 128
_NSUB = 16
_NPAD = 10240
_EMAIN = 19968
_ETAIL = _E - _NSUB * _EMAIN
_EC = _EMAIN + _ETAIL
_NC = _NPAD // _NSUB
_L = 16
_DUMMY = _NPAD - 1
_UNROLL = 8


def _rsqrt16(x):
    i = lax.bitcast_convert_type(x, jnp.int32)
    i = jnp.int32(0x5F3759DF) - (i >> 1)
    y = lax.bitcast_convert_type(i, jnp.float32)
    for _ in range(3):
        y = y * (jnp.float32(1.5) - jnp.float32(0.5) * x * y * y)
    return y


def _sc_node_weights(ei_hbm, w_hbm,
                     ei_v, deg_l, s_l, dis_l, red_v, node_w, slot_sp, dis_sp,
                     sem):
    wid = lax.axis_index("s")
    ebase = wid * _EMAIN
    nbase = wid * _NC
    ones = jnp.full((_L,), 1.0, jnp.float32)

    stage = pltpu.async_copy(ei_hbm.at[:, pl.ds(ebase, _EMAIN)],
                             ei_v.at[:, pl.ds(0, _EMAIN)], sem)

    @plsc.parallel_loop(0, _ETAIL // _L, unroll=8)
    def _fill_dummy(i):
        sl = pl.ds(_EMAIN + i * _L, _L)
        ei_v[0, sl] = jnp.full((_L,), _DUMMY, jnp.int32)
        ei_v[1, sl] = jnp.full((_L,), _DUMMY, jnp.int32)

    @plsc.parallel_loop(0, _NPAD // _L, unroll=8)
    def _zero(i):
        sl = pl.ds(i * _L, _L)
        deg_l[sl] = jnp.zeros((_L,), jnp.float32)
        s_l[sl] = jnp.zeros((_L,), jnp.float32)

    stage.wait()

    @pl.when(wid == _NSUB - 1)
    def _tail():
        pltpu.sync_copy(ei_hbm.at[:, pl.ds(_NSUB * _EMAIN, _ETAIL)],
                        ei_v.at[:, pl.ds(_EMAIN, _ETAIL)])

    @plsc.parallel_loop(0, _EC // _L, unroll=_UNROLL)
    def _hist(i):
        idx = ei_v[1, pl.ds(i * _L, _L)]
        plsc.addupdate_scatter(deg_l, [idx], ones)

    pltpu.sync_copy(deg_l, slot_sp.at[wid])
    plsc.subcore_barrier()
    pltpu.sync_copy(slot_sp.at[:, pl.ds(nbase, _NC)], red_v)

    @plsc.parallel_loop(0, _NC // _L, unroll=4)
    def _mkdis(c):
        sl = pl.ds(c * _L, _L)
        acc = jnp.full((_L,), 1.0, jnp.float32)
        for j in range(_NSUB):
            acc = acc + red_v[j, sl]
        node_w[sl] = _rsqrt16(acc)
    pltpu.sync_copy(node_w, dis_sp.at[pl.ds(nbase, _NC)])
    plsc.subcore_barrier()

    pltpu.sync_copy(dis_sp, dis_l)

    @plsc.parallel_loop(0, _EC // _L, unroll=_UNROLL)
    def _seg(i):
        d_idx = ei_v[1, pl.ds(i * _L, _L)]
        s_idx = ei_v[0, pl.ds(i * _L, _L)]
        g = plsc.load_gather(dis_l, [d_idx])
        plsc.addupdate_scatter(s_l, [s_idx], g)

    pltpu.sync_copy(s_l, slot_sp.at[wid])
    plsc.subcore_barrier()
    pltpu.sync_copy(slot_sp.at[:, pl.ds(nbase, _NC)], red_v)

    @plsc.parallel_loop(0, _NC // _L, unroll=4)
    def _wfin(c):
        sl = pl.ds(c * _L, _L)
        acc = red_v[0, sl]
        for j in range(1, _NSUB):
            acc = acc + red_v[j, sl]
        d = node_w[sl]
        node_w[sl] = d * (acc + d)
    pltpu.sync_copy(node_w, w_hbm.at[0, pl.ds(nbase, _NC)])


_sc_kernel = functools.partial(
    pl.kernel,
    out_type=jax.ShapeDtypeStruct((1, _NPAD), jnp.float32),
    mesh=plsc.VectorSubcoreMesh(core_axis_name="c", subcore_axis_name="s",
                                num_cores=1),
    compiler_params=pltpu.CompilerParams(needs_layout_passes=False),
    scratch_types=[
        pltpu.VMEM((2, _EC), jnp.int32),
        pltpu.VMEM((_NPAD,), jnp.float32),
        pltpu.VMEM((_NPAD,), jnp.float32),
        pltpu.VMEM((_NPAD,), jnp.float32),
        pltpu.VMEM((_NSUB, _NC), jnp.float32),
        pltpu.VMEM((_NC,), jnp.float32),
        pltpu.VMEM_SHARED((_NSUB, _NPAD), jnp.float32),
        pltpu.VMEM_SHARED((_NPAD,), jnp.float32),
        pltpu.SemaphoreType.DMA,
    ],
)(_sc_node_weights)


def _tc_a_body(x_ref, wa_ref, aa_ref, h1_ref):
    x = x_ref[...]
    wh = jnp.dot(x, wa_ref[...], preferred_element_type=jnp.float32)
    e = jnp.dot(wh, aa_ref[...], preferred_element_type=jnp.float32)
    e = jnp.where(e > 0, e, jnp.float32(0.2) * e)
    p = jnp.exp(e - jnp.max(e))
    attn = p * (jnp.float32(1.0) / jnp.sum(p))
    z = attn * wh
    h1 = jnp.where(z > 0, z, jnp.exp(z) - jnp.float32(1.0))
    h1_ref[...] = h1.astype(jnp.bfloat16)


def _tc_b_body(h1_ref, w_ref, wg_ref, bg_ref, wo_ref, bo_ref, o_ref):
    v = jnp.dot(w_ref[...][:, :_N].astype(jnp.bfloat16), h1_ref[...],
                preferred_element_type=jnp.float32)
    g = (jnp.dot(v, wg_ref[...], preferred_element_type=jnp.float32)
         * jnp.float32(1.0 / _N) + bg_ref[...])
    r = (jnp.dot(g, wo_ref[...], preferred_element_type=jnp.float32)
         + bo_ref[...])
    r = r - jnp.max(r)
    pr = jnp.exp(r)
    o_ref[...] = pr * (jnp.float32(1.0) / jnp.sum(pr))


def kernel(features1, edge_index1, edgesAttr1, adjacency1, node2node_features1,
           W_att, a_att, W_gcn, b_gcn, W_out, b_out):
    w_row = _sc_kernel(edge_index1)
    h1 = pl.pallas_call(
        _tc_a_body,
        out_shape=jax.ShapeDtypeStruct((_N, _D), jnp.bfloat16),
    )(features1, W_att, a_att)
    out = pl.pallas_call(
        _tc_b_body,
        out_shape=jax.ShapeDtypeStruct((1, 2), jnp.float32),
    )(h1, w_row, W_gcn, b_gcn.reshape(1, _D), W_out, b_out.reshape(1, 2))
    return out


# speedup vs baseline: 1.6679x; 1.0240x over previous
import functools

import jax
import jax.numpy as jnp
from jax import lax
from jax.experimental import pallas as pl
from jax.experimental.pallas import tpu as pltpu
from jax.experimental.pallas import tpu_sc as plsc

_N = 10000
_E = 320000
_D = 128
_NSUB = 16
_NPAD = 10240
_EMAIN = 19968
_ETAIL = _E - _NSUB * _EMAIN
_EC = _EMAIN + _ETAIL
_NC = _NPAD // _NSUB
_L = 16
_DUMMY = _NPAD - 1
_UNROLL = 8


def _rsqrt16(x):
    i = lax.bitcast_convert_type(x, jnp.int32)
    i = jnp.int32(0x5F3759DF) - (i >> 1)
    y = lax.bitcast_convert_type(i, jnp.float32)
    for _ in range(3):
        y = y * (jnp.float32(1.5) - jnp.float32(0.5) * x * y * y)
    return y


def _sc_node_weights(ei_hbm, dis_hbm, sparts_hbm,
                     ei_v, deg_l, s_l, dis_l, red_v, node_w, slot_sp, dis_sp,
                     sem):
    wid = lax.axis_index("s")
    ebase = wid * _EMAIN
    nbase = wid * _NC
    ones = jnp.full((_L,), 1.0, jnp.float32)

    stage = pltpu.async_copy(ei_hbm.at[:, pl.ds(ebase, _EMAIN)],
                             ei_v.at[:, pl.ds(0, _EMAIN)], sem)

    @plsc.parallel_loop(0, _ETAIL // _L, unroll=8)
    def _fill_dummy(i):
        sl = pl.ds(_EMAIN + i * _L, _L)
        ei_v[0, sl] = jnp.full((_L,), _DUMMY, jnp.int32)
        ei_v[1, sl] = jnp.full((_L,), _DUMMY, jnp.int32)

    @plsc.parallel_loop(0, _NPAD // _L, unroll=8)
    def _zero(i):
        sl = pl.ds(i * _L, _L)
        deg_l[sl] = jnp.zeros((_L,), jnp.float32)
        s_l[sl] = jnp.zeros((_L,), jnp.float32)

    stage.wait()

    @pl.when(wid == _NSUB - 1)
    def _tail():
        pltpu.sync_copy(ei_hbm.at[:, pl.ds(_NSUB * _EMAIN, _ETAIL)],
                        ei_v.at[:, pl.ds(_EMAIN, _ETAIL)])

    @plsc.parallel_loop(0, _EC // _L, unroll=_UNROLL)
    def _hist(i):
        idx = ei_v[1, pl.ds(i * _L, _L)]
        plsc.addupdate_scatter(deg_l, [idx], ones)

    pltpu.sync_copy(deg_l, slot_sp.at[wid])
    plsc.subcore_barrier()
    pltpu.sync_copy(slot_sp.at[:, pl.ds(nbase, _NC)], red_v)

    @plsc.parallel_loop(0, _NC // _L, unroll=4)
    def _mkdis(c):
        sl = pl.ds(c * _L, _L)
        acc = jnp.full((_L,), 1.0, jnp.float32)
        for j in range(_NSUB):
            acc = acc + red_v[j, sl]
        node_w[sl] = _rsqrt16(acc)
    pltpu.sync_copy(node_w, dis_sp.at[pl.ds(nbase, _NC)])
    pltpu.sync_copy(node_w, dis_hbm.at[0, pl.ds(nbase, _NC)])
    plsc.subcore_barrier()

    pltpu.sync_copy(dis_sp, dis_l)

    @plsc.parallel_loop(0, _EC // _L, unroll=_UNROLL)
    def _seg(i):
        d_idx = ei_v[1, pl.ds(i * _L, _L)]
        s_idx = ei_v[0, pl.ds(i * _L, _L)]
        g = plsc.load_gather(dis_l, [d_idx])
        plsc.addupdate_scatter(s_l, [s_idx], g)

    pltpu.sync_copy(s_l, sparts_hbm.at[wid])


_sc_kernel = functools.partial(
    pl.kernel,
    out_type=(jax.ShapeDtypeStruct((1, _NPAD), jnp.float32),
              jax.ShapeDtypeStruct((_NSUB, _NPAD), jnp.float32)),
    mesh=plsc.VectorSubcoreMesh(core_axis_name="c", subcore_axis_name="s",
                                num_cores=1),
    compiler_params=pltpu.CompilerParams(needs_layout_passes=False),
    scratch_types=[
        pltpu.VMEM((2, _EC), jnp.int32),
        pltpu.VMEM((_NPAD,), jnp.float32),
        pltpu.VMEM((_NPAD,), jnp.float32),
        pltpu.VMEM((_NPAD,), jnp.float32),
        pltpu.VMEM((_NSUB, _NC), jnp.float32),
        pltpu.VMEM((_NC,), jnp.float32),
        pltpu.VMEM_SHARED((_NSUB, _NPAD), jnp.float32),
        pltpu.VMEM_SHARED((_NPAD,), jnp.float32),
        pltpu.SemaphoreType.DMA,
    ],
)(_sc_node_weights)


def _tc_a_body(x_ref, wa_ref, aa_ref, h1_ref):
    x = x_ref[...]
    wh = jnp.dot(x, wa_ref[...], preferred_element_type=jnp.float32)
    e = jnp.dot(wh, aa_ref[...], preferred_element_type=jnp.float32)
    e = jnp.where(e > 0, e, jnp.float32(0.2) * e)
    p = jnp.exp(e - jnp.max(e))
    attn = p * (jnp.float32(1.0) / jnp.sum(p))
    z = attn * wh
    h1 = jnp.where(z > 0, z, jnp.exp(z) - jnp.float32(1.0))
    h1_ref[...] = h1.astype(jnp.bfloat16)


def _tc_b_body(h1_ref, dis_ref, sp_ref, wg_ref, bg_ref, wo_ref, bo_ref,
               o_ref):
    dis = dis_ref[...]
    s = jnp.sum(sp_ref[...], axis=0, keepdims=True)
    w = dis * (s + dis)
    v = jnp.dot(w[:, :_N].astype(jnp.bfloat16), h1_ref[...],
                preferred_element_type=jnp.float32)
    g = (jnp.dot(v, wg_ref[...], preferred_element_type=jnp.float32)
         * jnp.float32(1.0 / _N) + bg_ref[...])
    r = (jnp.dot(g, wo_ref[...], preferred_element_type=jnp.float32)
         + bo_ref[...])
    r = r - jnp.max(r)
    pr = jnp.exp(r)
    o_ref[...] = pr * (jnp.float32(1.0) / jnp.sum(pr))


def kernel(features1, edge_index1, edgesAttr1, adjacency1, node2node_features1,
           W_att, a_att, W_gcn, b_gcn, W_out, b_out):
    dis_row, s_parts = _sc_kernel(edge_index1)
    h1 = pl.pallas_call(
        _tc_a_body,
        out_shape=jax.ShapeDtypeStruct((_N, _D), jnp.bfloat16),
    )(features1, W_att, a_att)
    out = pl.pallas_call(
        _tc_b_body,
        out_shape=jax.ShapeDtypeStruct((1, 2), jnp.float32),
    )(h1, dis_row, s_parts, W_gcn, b_gcn.reshape(1, _D), W_out,
      b_out.reshape(1, 2))
    return out
